# initial kernel scaffold (unmeasured)
import jax
import jax.numpy as jnp
from jax import lax
from jax.experimental import pallas as pl
from jax.experimental.pallas import tpu as pltpu

N_DEV = 4


def _layer_body(x_ref, win_ref, wout_ref, out_ref,
                send_ref, comm_ref, send_sems, recv_sems):
    my = lax.axis_index("i")

    barrier_sem = pltpu.get_barrier_semaphore()
    for off in (1, 2, 3):
        peer = lax.rem(my + off, N_DEV)
        pl.semaphore_signal(
            barrier_sem, inc=1,
            device_id=(peer,), device_id_type=pl.DeviceIdType.MESH,
        )
    pl.semaphore_wait(barrier_sem, 3)

    partial = jnp.dot(
        x_ref[...].astype(jnp.bfloat16),
        win_ref[...].astype(jnp.bfloat16),
        preferred_element_type=jnp.float32,
    )
    send_ref[...] = partial.astype(jnp.bfloat16)

    rdmas = []
    for off in (1, 2, 3):
        peer = lax.rem(my + off, N_DEV)
        rdma = pltpu.make_async_remote_copy(
            src_ref=send_ref,
            dst_ref=comm_ref.at[off - 1],
            send_sem=send_sems.at[off - 1],
            recv_sem=recv_sems.at[off - 1],
            device_id=(peer,),
            device_id_type=pl.DeviceIdType.MESH,
        )
        rdma.start()
        rdmas.append(rdma)

    acc = partial
    for slot, rdma in enumerate(rdmas):
        rdma.wait_recv()
        acc = acc + comm_ref[slot].astype(jnp.float32)
    for rdma in rdmas:
        rdma.wait_send()

    h = jnp.maximum(acc, 0.0).astype(jnp.bfloat16)
    out_ref[...] = jnp.dot(
        h, wout_ref[...].astype(jnp.bfloat16),
        preferred_element_type=jnp.float32,
    )


def _layer(x, win, wout, cid):
    b = x.shape[0]
    h_dim = win.shape[1]
    d_out = wout.shape[1]
    return pl.pallas_call(
        _layer_body,
        out_shape=jax.ShapeDtypeStruct((b, d_out), jnp.float32),
        in_specs=[
            pl.BlockSpec(memory_space=pltpu.VMEM),
            pl.BlockSpec(memory_space=pltpu.VMEM),
            pl.BlockSpec(memory_space=pltpu.VMEM),
        ],
        out_specs=pl.BlockSpec(memory_space=pltpu.VMEM),
        scratch_shapes=[
            pltpu.VMEM((b, h_dim), jnp.bfloat16),
            pltpu.VMEM((3, b, h_dim), jnp.bfloat16),
            pltpu.SemaphoreType.DMA((3,)),
            pltpu.SemaphoreType.DMA((3,)),
        ],
        compiler_params=pltpu.CompilerParams(collective_id=cid),
    )(x, win, wout)


def kernel(x, Win0, Wout0, Win1, Wout1, Win2, Wout2):
    x = _layer(x, Win0, Wout0, cid=0)
    x = _layer(x, Win1, Wout1, cid=1)
    x = _layer(x, Win2, Wout2, cid=2)
    return x


# baseline (device time: 130985 ns/iter reference)
import jax
import jax.numpy as jnp
from jax import lax
from jax.experimental import pallas as pl
from jax.experimental.pallas import tpu as pltpu

N_DEV = 4
NH = 8


def _layer_body(x_ref, win_ref, wout_ref, out_ref,
                send_ref, comm_ref, send_sems, recv_sems):
    j = pl.program_id(0)
    my = lax.axis_index("i")

    @pl.when(j == 0)
    def _():
        barrier_sem = pltpu.get_barrier_semaphore()
        for off in (1, 2, 3):
            peer = lax.rem(my + off, N_DEV)
            pl.semaphore_signal(
                barrier_sem, inc=1,
                device_id=(peer,), device_id_type=pl.DeviceIdType.MESH,
            )
        pl.semaphore_wait(barrier_sem, 3)
        out_ref[...] = jnp.zeros_like(out_ref)

    partial = jnp.dot(
        x_ref[...].astype(jnp.bfloat16),
        win_ref[...].astype(jnp.bfloat16),
        preferred_element_type=jnp.float32,
    )
    send_ref[j] = partial.astype(jnp.bfloat16)

    rdmas = []
    for off in (1, 2, 3):
        peer = lax.rem(my + off, N_DEV)
        rdma = pltpu.make_async_remote_copy(
            src_ref=send_ref.at[j],
            dst_ref=comm_ref.at[j, off - 1],
            send_sem=send_sems.at[j, off - 1],
            recv_sem=recv_sems.at[j, off - 1],
            device_id=(peer,),
            device_id_type=pl.DeviceIdType.MESH,
        )
        rdma.start()
        rdmas.append(rdma)

    acc = partial
    for slot, rdma in enumerate(rdmas):
        rdma.wait_recv()
        acc = acc + comm_ref[j, slot].astype(jnp.float32)

    h = jnp.maximum(acc, 0.0).astype(jnp.bfloat16)
    out_ref[...] += jnp.dot(
        h, wout_ref[...].astype(jnp.bfloat16),
        preferred_element_type=jnp.float32,
    )

    for rdma in rdmas:
        rdma.wait_send()


def _layer(x, win, wout, cid):
    b, d_in = x.shape
    h_dim = win.shape[1]
    d_out = wout.shape[1]
    hc = h_dim // NH
    return pl.pallas_call(
        _layer_body,
        grid=(NH,),
        out_shape=jax.ShapeDtypeStruct((b, d_out), jnp.float32),
        in_specs=[
            pl.BlockSpec((b, d_in), lambda j: (0, 0),
                         memory_space=pltpu.VMEM),
            pl.BlockSpec((d_in, hc), lambda j: (0, j),
                         memory_space=pltpu.VMEM),
            pl.BlockSpec((hc, d_out), lambda j: (j, 0),
                         memory_space=pltpu.VMEM),
        ],
        out_specs=pl.BlockSpec((b, d_out), lambda j: (0, 0),
                               memory_space=pltpu.VMEM),
        scratch_shapes=[
            pltpu.VMEM((NH, b, hc), jnp.bfloat16),
            pltpu.VMEM((NH, 3, b, hc), jnp.bfloat16),
            pltpu.SemaphoreType.DMA((NH, 3)),
            pltpu.SemaphoreType.DMA((NH, 3)),
        ],
        compiler_params=pltpu.CompilerParams(
            collective_id=cid,
            dimension_semantics=("arbitrary",),
        ),
    )(x, win, wout)


def kernel(x, Win0, Wout0, Win1, Wout1, Win2, Wout2):
    x = _layer(x, Win0, Wout0, cid=0)
    x = _layer(x, Win1, Wout1, cid=1)
    x = _layer(x, Win2, Wout2, cid=2)
    return x
